# baseline (speedup 1.0000x reference)
import functools
import jax
import jax.numpy as jnp
from jax import lax
from jax.experimental import pallas as pl
from jax.experimental.pallas import tpu as pltpu
from jax.experimental.pallas import tpu_sc as plsc

NC = 2
NS = 16
NW = NC * NS
CB = 128


@jax.jit
def kernel(token_ids, table):
    B, L = token_ids.shape
    V, D = table.shape
    assert D == 32 and B % (NW * CB) == 0
    DT, DR = D // 8, 8
    cb_per_w = B // (NW * CB)
    toks_per_w = cb_per_w * CB * L
    chunks = cb_per_w * L
    idx_flat = token_ids.reshape(B * L // CB, CB).astype(jnp.int32)

    mesh = plsc.VectorSubcoreMesh(
        core_axis_name="c", subcore_axis_name="s",
        num_cores=NC, num_subcores=NS)

    @functools.partial(
        pl.kernel,
        mesh=mesh,
        out_type=jax.ShapeDtypeStruct((L, DT, B // CB, DR, CB), jnp.float32),
        scratch_types=[
            pltpu.VMEM((toks_per_w // CB, CB), jnp.int32),
            pltpu.VMEM((CB,), jnp.int32),
            pltpu.VMEM((CB,), jnp.int32),
            pltpu.VMEM((CB, D), jnp.float32),
            pltpu.VMEM((CB, D), jnp.float32),
            pltpu.VMEM((DT, DR, CB), jnp.float32),
            pltpu.SemaphoreType.DMA,
            pltpu.SemaphoreType.DMA,
        ],
        compiler_params=pltpu.CompilerParams(
            use_tc_tiling_on_sc=False, needs_layout_passes=False),
    )
    def emb(table_hbm, idx_hbm, out_hbm, idx_v, ic_a, ic_b, g_a, g_b,
            tbuf, gsem_a, gsem_b):
        wid = lax.axis_index("s") * NC + lax.axis_index("c")
        pltpu.sync_copy(
            idx_hbm.at[pl.ds(wid * (toks_per_w // CB), toks_per_w // CB)],
            idx_v)
        lanes = lax.iota(jnp.int32, 16)
        svecs = [lanes + (k * 16) for k in range(8)]
        lanes_l = lanes * L

        def build_idx(m, ic):
            base = (m // L) * (CB * L) + (m % L)

            @plsc.parallel_loop(0, 8, unroll=8)
            def _(k):
                p = lanes_l + (k * (16 * L) + base)
                ic[pl.ds(k * 16, 16)] = plsc.load_gather(
                    idx_v, [p >> 7, p & (CB - 1)])

        def fire(ic, g, sem):
            pltpu.async_copy(table_hbm.at[ic], g, sem)

        def wait(ic, g, sem):
            pltpu.make_async_copy(table_hbm.at[ic], g, sem).wait()

        def emit(m, g):
            @plsc.parallel_loop(0, D, unroll=8)
            def _(d):
                dvec = jnp.full((16,), d, jnp.int32)
                for k in range(8):
                    tbuf[d >> 3, d & 7, pl.ds(k * 16, 16)] = plsc.load_gather(
                        g, [svecs[k], dvec])
            c = wid * cb_per_w + m // L
            pltpu.sync_copy(tbuf, out_hbm.at[m % L, :, c])

        build_idx(0, ic_a)
        fire(ic_a, g_a, gsem_a)

        @pl.loop(0, chunks, step=2)
        def _(m):
            build_idx(m + 1, ic_b)
            fire(ic_b, g_b, gsem_b)
            wait(ic_a, g_a, gsem_a)
            emit(m, g_a)

            @pl.when(m + 2 < chunks)
            def _():
                build_idx(m + 2, ic_a)
                fire(ic_a, g_a, gsem_a)

            wait(ic_b, g_b, gsem_b)
            emit(m + 1, g_b)

    out5 = emb(table, idx_flat)
    return out5.transpose(2, 4, 0, 1, 3).reshape(B, L, D)


# speedup vs baseline: 1.0761x; 1.0761x over previous
import functools
import jax
import jax.numpy as jnp
from jax import lax
from jax.experimental import pallas as pl
from jax.experimental.pallas import tpu as pltpu
from jax.experimental.pallas import tpu_sc as plsc

NC = 2
NS = 16
NW = NC * NS
CB = 128


@jax.jit
def kernel(token_ids, table):
    B, L = token_ids.shape
    V, D = table.shape
    assert D == 32 and B % (NW * CB) == 0
    DT, DR = D // 8, 8
    cb_per_w = B // (NW * CB)
    toks_per_w = cb_per_w * CB * L
    chunks = cb_per_w * L
    idx_flat = token_ids.reshape(B * L // CB, CB).astype(jnp.int32)

    mesh = plsc.VectorSubcoreMesh(
        core_axis_name="c", subcore_axis_name="s",
        num_cores=NC, num_subcores=NS)

    @functools.partial(
        pl.kernel,
        mesh=mesh,
        out_type=jax.ShapeDtypeStruct((L, DT, B // CB, DR, CB), jnp.float32),
        scratch_types=[
            pltpu.VMEM((toks_per_w // CB, CB), jnp.int32),
            pltpu.VMEM((CB,), jnp.int32),
            pltpu.VMEM((CB,), jnp.int32),
            pltpu.VMEM((CB, D), jnp.float32),
            pltpu.VMEM((CB, D), jnp.float32),
            pltpu.VMEM((DT, DR, CB), jnp.float32),
            pltpu.VMEM((DT, DR, CB), jnp.float32),
            pltpu.SemaphoreType.DMA,
            pltpu.SemaphoreType.DMA,
            pltpu.SemaphoreType.DMA,
            pltpu.SemaphoreType.DMA,
        ],
        compiler_params=pltpu.CompilerParams(
            use_tc_tiling_on_sc=False, needs_layout_passes=False),
    )
    def emb(table_hbm, idx_hbm, out_hbm, idx_v, ic_a, ic_b, g_a, g_b,
            t_a, t_b, gsem_a, gsem_b, osem_a, osem_b):
        wid = lax.axis_index("s") * NC + lax.axis_index("c")
        pltpu.sync_copy(
            idx_hbm.at[pl.ds(wid * (toks_per_w // CB), toks_per_w // CB)],
            idx_v)
        lanes = lax.iota(jnp.int32, 16)
        svecs = [lanes + (k * 16) for k in range(8)]
        lanes_l = lanes * L

        def build_idx(m, ic):
            base = (m // L) * (CB * L) + (m % L)

            @plsc.parallel_loop(0, 8, unroll=8)
            def _(k):
                p = lanes_l + (k * (16 * L) + base)
                ic[pl.ds(k * 16, 16)] = plsc.load_gather(
                    idx_v, [p >> 7, p & (CB - 1)])

        def fire(ic, g, sem):
            pltpu.async_copy(table_hbm.at[ic], g, sem)

        def wait(ic, g, sem):
            pltpu.make_async_copy(table_hbm.at[ic], g, sem).wait()

        def out_at(m):
            return out_hbm.at[m % L, :, wid * cb_per_w + m // L]

        def emit(m, g, tb, osem):
            @plsc.parallel_loop(0, D, unroll=4)
            def _(d):
                dvec = jnp.full((16,), d, jnp.int32)
                for k in range(8):
                    tb[d >> 3, d & 7, pl.ds(k * 16, 16)] = plsc.load_gather(
                        g, [svecs[k], dvec])

            @pl.when(m >= 2)
            def _():
                pltpu.make_async_copy(tb, out_at(m - 2), osem).wait()

            pltpu.async_copy(tb, out_at(m), osem)

        build_idx(0, ic_a)
        fire(ic_a, g_a, gsem_a)

        @pl.loop(0, chunks, step=2)
        def _(m):
            build_idx(m + 1, ic_b)
            fire(ic_b, g_b, gsem_b)
            wait(ic_a, g_a, gsem_a)
            emit(m, g_a, t_a, osem_a)

            @pl.when(m + 2 < chunks)
            def _():
                build_idx(m + 2, ic_a)
                fire(ic_a, g_a, gsem_a)

            wait(ic_b, g_b, gsem_b)
            emit(m + 1, g_b, t_b, osem_b)

        pltpu.make_async_copy(t_a, out_at(chunks - 2), osem_a).wait()
        pltpu.make_async_copy(t_b, out_at(chunks - 1), osem_b).wait()

    out5 = emb(table, idx_flat)
    return out5.transpose(2, 4, 0, 1, 3).reshape(B, L, D)
